# Initial kernel scaffold; baseline (speedup 1.0000x reference)
import jax
import jax.numpy as jnp
from jax.experimental import pallas as pl


def kernel(input, target):
    raise NotImplementedError("write your pallas kernel here")


# speedup vs baseline: 21.4397x; 21.4397x over previous
import functools

import jax
import jax.numpy as jnp
from jax import lax
from jax.experimental import pallas as pl
from jax.experimental.pallas import tpu as pltpu
from jax.experimental.pallas import tpu_sc as plsc

B, C, H, W = 4, 19, 512, 512
N = H * W
NTASK = B * C
K = 8192
CH = 16384
NCHUNK = N // CH
NW = 32
L = 16
TASK_ROUNDS = (NTASK + NW - 1) // NW


def _tc_errors_kernel(x_ref, t_ref, e_ref, tf_ref):
    x = x_ref[...]
    t = t_ref[...]
    m = jnp.max(x, axis=1, keepdims=True)
    ex = jnp.exp(x - m)
    p = ex / jnp.sum(ex, axis=1, keepdims=True)
    cids = lax.broadcasted_iota(jnp.int32, (1, C, 1), 1)
    fg = (t == cids).astype(jnp.float32)
    e_ref[...] = jnp.abs(p - fg)
    tf_ref[...] = t.astype(jnp.float32)


def _sc_hist_kernel(e_hbm, t_hbm, out_hbm, e_buf, t_buf,
                    hist_n, hist_s, hist_e, accv):
    cid = lax.axis_index("c")
    sid = lax.axis_index("s")
    wid = sid * 2 + cid

    zeros16 = jnp.zeros((L,), jnp.float32)
    ones16 = jnp.ones((L,), jnp.float32)
    accv[...] = zeros16

    for i in range(TASK_ROUNDS):
        tid = wid + NW * i

        @pl.when(tid < NTASK)
        def _():
            b = ((tid >= C).astype(jnp.int32)
                 + (tid >= 2 * C).astype(jnp.int32)
                 + (tid >= 3 * C).astype(jnp.int32))

            def zbody(j, _):
                sl = pl.ds(j * L, L)
                hist_n[sl] = zeros16
                hist_s[sl] = zeros16
                hist_e[sl] = zeros16
                return 0
            lax.fori_loop(0, K // L, zbody, 0)

            def hbody(j, tacc):
                sl = pl.ds(j * L, L)
                ev = e_buf[sl]
                tv = t_buf[sl]
                qi = ((1.0 - ev) * float(K)).astype(jnp.int32)
                qi = jnp.minimum(jnp.maximum(qi, 0), K - 1)
                plsc.addupdate_scatter(hist_n, [qi], ones16)
                plsc.addupdate_scatter(hist_s, [qi], tv)
                plsc.addupdate_scatter(hist_e, [qi], ev)
                return tacc + tv

            tacc = zeros16
            for ci in range(NCHUNK):
                pltpu.sync_copy(e_hbm.at[pl.ds(tid * N + ci * CH, CH)], e_buf)
                pltpu.sync_copy(t_hbm.at[pl.ds(b * N + ci * CH, CH)], t_buf)
                tacc = lax.fori_loop(0, CH // L, hbody, tacc)
            T = jnp.sum(tacc)

            def sbody(j, carry):
                kc, Sc, acc = carry
                sl = pl.ds(j * L, L)
                nv = hist_n[sl]
                sv = hist_s[sl]
                Ev = hist_e[sl]
                kv = kc + plsc.cumsum(nv)
                Sv = Sc + plsc.cumsum(sv)
                den1 = T + kv - Sv
                denp = den1 - nv + sv
                num = nv * (T - Sv) + kv * sv
                term = Ev * num / (jnp.maximum(nv, 1.0)
                                   * jnp.maximum(den1 * denp, 1.0))
                return (kc + jnp.sum(nv), Sc + jnp.sum(sv), acc + term)

            _, _, acc = lax.fori_loop(
                0, K // L, sbody,
                (jnp.float32(0.0), jnp.float32(0.0), zeros16))
            accv[...] = accv[...] + acc

    pltpu.sync_copy(accv, out_hbm.at[wid])


@jax.jit
def kernel(input, target):
    x = input.reshape(B, C, N)
    t3 = target.astype(jnp.int32).reshape(B, 1, N)

    CH1 = 32768
    e, tf = pl.pallas_call(
        _tc_errors_kernel,
        grid=(B, N // CH1),
        in_specs=[
            pl.BlockSpec((1, C, CH1), lambda b, j: (b, 0, j)),
            pl.BlockSpec((1, 1, CH1), lambda b, j: (b, 0, j)),
        ],
        out_specs=[
            pl.BlockSpec((1, C, CH1), lambda b, j: (b, 0, j)),
            pl.BlockSpec((1, 1, CH1), lambda b, j: (b, 0, j)),
        ],
        out_shape=[
            jax.ShapeDtypeStruct((B, C, N), jnp.float32),
            jax.ShapeDtypeStruct((B, 1, N), jnp.float32),
        ],
    )(x, t3)

    sc_fn = pl.kernel(
        _sc_hist_kernel,
        out_type=jax.ShapeDtypeStruct((NW, L), jnp.float32),
        mesh=plsc.VectorSubcoreMesh(core_axis_name="c", subcore_axis_name="s"),
        compiler_params=pltpu.CompilerParams(needs_layout_passes=False),
        scratch_types=[
            pltpu.VMEM((CH,), jnp.float32),
            pltpu.VMEM((CH,), jnp.float32),
            pltpu.VMEM((K,), jnp.float32),
            pltpu.VMEM((K,), jnp.float32),
            pltpu.VMEM((K,), jnp.float32),
            pltpu.VMEM((L,), jnp.float32),
        ],
    )
    partial = sc_fn(e.reshape(B * C * N), tf.reshape(B * N))
    return jnp.sum(partial) / (B * C)


# speedup vs baseline: 40.5756x; 1.8925x over previous
import jax
import jax.numpy as jnp
from jax import lax
from jax.experimental import pallas as pl
from jax.experimental.pallas import tpu as pltpu
from jax.experimental.pallas import tpu_sc as plsc

B, C, H, W = 4, 19, 512, 512
N = H * W
NTASK = B * C
K = 2048
NSLOT = 16
CH = 16384
NCHUNK = N // CH
NW = 32
L = 16
TASK_ROUNDS = (NTASK + NW - 1) // NW


def _tc_pack_kernel(x_ref, t_ref, pk_ref):
    x = x_ref[...]
    t = t_ref[...]
    m = jnp.max(x, axis=1, keepdims=True)
    ex = jnp.exp(x - m)
    p = ex / jnp.sum(ex, axis=1, keepdims=True)
    cids = lax.broadcasted_iota(jnp.int32, (1, C, 1), 1)
    fg = (t == cids).astype(jnp.float32)
    e = jnp.abs(p - fg)
    qi = ((1.0 - e) * float(K)).astype(jnp.int32)
    qi = jnp.minimum(jnp.maximum(qi, 0), K - 1)
    pk_ref[...] = (qi | (t << 11)).astype(jnp.int16)


def _sc_hist_kernel(pk_hbm, out_hbm, pk_buf0, pk_buf1, hist_n, hist_s,
                    fold_n, fold_s, accv, sem0, sem1):
    cid = lax.axis_index("c")
    sid = lax.axis_index("s")
    wid = sid * 2 + cid

    zeros16 = jnp.zeros((L,), jnp.float32)
    ones16 = jnp.ones((L,), jnp.float32)
    lane = lax.broadcasted_iota(jnp.int32, (L,), 0)
    sems = (sem0, sem1)
    accv[...] = zeros16

    for i in range(TASK_ROUNDS):
        tid = wid + NW * i

        @pl.when(tid < NTASK)
        def _():
            def zbody(j, _):
                sl = pl.ds(j * L, L)
                hist_n[sl] = zeros16
                hist_s[sl] = zeros16
                return 0
            lax.fori_loop(0, K * NSLOT // L, zbody, 0)

            bufs = (pk_buf0, pk_buf1)

            def do_chunk(ci):
                pkb = bufs[ci % 2]

                @plsc.parallel_loop(0, CH // 32, unroll=4)
                def _hist(j):
                    v = plsc.bitcast(pkb[pl.ds(j * 32, 32)], jnp.int32)
                    for half in range(2):
                        if half == 0:
                            u = v & 0xFFFF
                        else:
                            u = lax.shift_right_logical(v, 16)
                        qi = u & (K - 1)
                        tv = lax.shift_right_logical(u, 11)
                        slot = (qi << 4) | lane
                        plsc.addupdate_scatter(hist_n, [slot], ones16)
                        plsc.addupdate_scatter(hist_s, [slot],
                                               tv.astype(jnp.float32))

            cp = pltpu.async_copy(
                pk_hbm.at[pl.ds(tid * N, CH)], pk_buf0, sem0)
            for ci in range(NCHUNK):
                if ci + 1 < NCHUNK:
                    nxt = pltpu.async_copy(
                        pk_hbm.at[pl.ds(tid * N + (ci + 1) * CH, CH)],
                        bufs[(ci + 1) % 2], sems[(ci + 1) % 2])
                cp.wait()
                do_chunk(ci)
                if ci + 1 < NCHUNK:
                    cp = nxt

            def fbody(g, tacc):
                base = ((g * L + lane) << 4)
                nacc = zeros16
                sacc = zeros16
                for sub in range(NSLOT):
                    idx = base + ((lane + sub) & (NSLOT - 1))
                    nacc = nacc + plsc.load_gather(hist_n, [idx])
                    sacc = sacc + plsc.load_gather(hist_s, [idx])
                sl = pl.ds(g * L, L)
                fold_n[sl] = nacc
                fold_s[sl] = sacc
                return tacc + sacc
            tacc = lax.fori_loop(0, K // L, fbody, zeros16)
            T = jnp.sum(tacc)

            def sbody(g, carry):
                kc, Sc, acc = carry
                sl = pl.ds(g * L, L)
                nv = fold_n[sl]
                sv = fold_s[sl]
                kv = kc + plsc.cumsum(nv)
                Sv = Sc + plsc.cumsum(sv)
                den1 = T + kv - Sv
                denp = den1 - nv + sv
                num = nv * (T - Sv) + kv * sv
                bf = (g * L + lane).astype(jnp.float32)
                mid = (float(K) - 0.5 - bf) * (1.0 / float(K))
                term = mid * num / jnp.maximum(den1 * denp, 1.0)
                return (kc + jnp.sum(nv), Sc + jnp.sum(sv), acc + term)

            _, _, acc = lax.fori_loop(
                0, K // L, sbody,
                (jnp.float32(0.0), jnp.float32(0.0), zeros16))
            accv[...] = accv[...] + acc

    pltpu.sync_copy(accv, out_hbm.at[wid])


@jax.jit
def kernel(input, target):
    x = input.reshape(B, C, N)
    t3 = target.astype(jnp.int32).reshape(B, 1, N)

    CH1 = 32768
    pk = pl.pallas_call(
        _tc_pack_kernel,
        grid=(B, N // CH1),
        in_specs=[
            pl.BlockSpec((1, C, CH1), lambda b, j: (b, 0, j)),
            pl.BlockSpec((1, 1, CH1), lambda b, j: (b, 0, j)),
        ],
        out_specs=pl.BlockSpec((1, C, CH1), lambda b, j: (b, 0, j)),
        out_shape=jax.ShapeDtypeStruct((B, C, N), jnp.int16),
    )(x, t3)

    sc_fn = pl.kernel(
        _sc_hist_kernel,
        out_type=jax.ShapeDtypeStruct((NW, L), jnp.float32),
        mesh=plsc.VectorSubcoreMesh(core_axis_name="c", subcore_axis_name="s"),
        compiler_params=pltpu.CompilerParams(needs_layout_passes=False),
        scratch_types=[
            pltpu.VMEM((CH,), jnp.int16),
            pltpu.VMEM((CH,), jnp.int16),
            pltpu.VMEM((K * NSLOT,), jnp.float32),
            pltpu.VMEM((K * NSLOT,), jnp.float32),
            pltpu.VMEM((K,), jnp.float32),
            pltpu.VMEM((K,), jnp.float32),
            pltpu.VMEM((L,), jnp.float32),
            pltpu.SemaphoreType.DMA,
            pltpu.SemaphoreType.DMA,
        ],
    )
    partial = sc_fn(pk.reshape(B * C * N))
    return jnp.sum(partial) / (B * C)


# speedup vs baseline: 107.0779x; 2.6390x over previous
import jax
import jax.numpy as jnp
from jax import lax
from jax.experimental import pallas as pl
from jax.experimental.pallas import tpu as pltpu
from jax.experimental.pallas import tpu_sc as plsc

B, C, H, W = 4, 19, 512, 512
N = H * W
NTASK = B * C
K = 2048
NSLOT = 16
CH = 16384
CROWS = CH // 128
NCHUNK = N // CH
TROWS = N // 128
NW = 32
L = 16
TASK_ROUNDS = (NTASK + NW - 1) // NW


def _tc_lse_kernel(x_ref, lse_ref):
    x = x_ref[...]
    m = jnp.max(x, axis=1, keepdims=True)
    lse_ref[...] = m + jnp.log(
        jnp.sum(jnp.exp(x - m), axis=1, keepdims=True))


def _tc_pack_kernel(x_ref, lse_ref, t_ref, pk_ref):
    c = pl.program_id(1)
    x = x_ref[...]
    lse = lse_ref[...]
    t = t_ref[...]
    p = jnp.exp(x - lse)
    fg = (t == c).astype(jnp.float32)
    e = jnp.abs(p - fg)
    qi = ((1.0 - e) * float(K)).astype(jnp.int32)
    qi = jnp.minimum(jnp.maximum(qi, 0), K - 1)
    pk_ref[...] = (qi | (t << 11)).astype(jnp.int16).reshape(H, 128)


def _sc_hist_kernel(pk_hbm, out_hbm, pk_buf0, pk_buf1, hist_n, hist_s,
                    fold_n, fold_s, accv, sem0, sem1):
    cid = lax.axis_index("c")
    sid = lax.axis_index("s")
    wid = sid * 2 + cid

    zeros16 = jnp.zeros((L,), jnp.float32)
    ones16 = jnp.ones((L,), jnp.float32)
    lane = lax.broadcasted_iota(jnp.int32, (L,), 0)
    sems = (sem0, sem1)
    accv[...] = zeros16

    for i in range(TASK_ROUNDS):
        tid = wid + NW * i

        @pl.when(tid < NTASK)
        def _():
            def zbody(j, _):
                sl = pl.ds(j * L, L)
                hist_n[sl] = zeros16
                hist_s[sl] = zeros16
                return 0
            lax.fori_loop(0, K * NSLOT // L, zbody, 0)

            bufs = (pk_buf0, pk_buf1)

            def do_chunk(ci):
                pkb = bufs[ci % 2]

                @plsc.parallel_loop(0, CH // 32, unroll=4)
                def _hist(j):
                    r = lax.shift_right_logical(j, 2)
                    cc = j & 3
                    v16 = pkb[r, pl.ds(cc * 32, 32)]
                    v = plsc.bitcast(v16, jnp.int32)
                    for half in range(2):
                        if half == 0:
                            u = v & 0xFFFF
                        else:
                            u = lax.shift_right_logical(v, 16)
                        qi = u & (K - 1)
                        tv = lax.shift_right_logical(u, 11) & 31
                        slot = (qi << 4) | lane
                        plsc.addupdate_scatter(hist_n, [slot], ones16)
                        plsc.addupdate_scatter(hist_s, [slot],
                                               tv.astype(jnp.float32))

            cp = pltpu.async_copy(
                pk_hbm.at[pl.ds(tid * TROWS, CROWS), :], pk_buf0, sem0)
            for ci in range(NCHUNK):
                if ci + 1 < NCHUNK:
                    nxt = pltpu.async_copy(
                        pk_hbm.at[pl.ds(tid * TROWS + (ci + 1) * CROWS,
                                        CROWS), :],
                        (pk_buf0, pk_buf1)[(ci + 1) % 2],
                        sems[(ci + 1) % 2])
                cp.wait()
                do_chunk(ci)
                if ci + 1 < NCHUNK:
                    cp = nxt

            def fbody(g, tacc):
                base = ((g * L + lane) << 4)
                nacc = zeros16
                sacc = zeros16
                for sub in range(NSLOT):
                    idx = base + ((lane + sub) & (NSLOT - 1))
                    nacc = nacc + plsc.load_gather(hist_n, [idx])
                    sacc = sacc + plsc.load_gather(hist_s, [idx])
                sl = pl.ds(g * L, L)
                fold_n[sl] = nacc
                fold_s[sl] = sacc
                return tacc + sacc
            tacc = lax.fori_loop(0, K // L, fbody, zeros16)
            T = jnp.sum(tacc)

            def sbody(g, carry):
                kc, Sc, acc = carry
                sl = pl.ds(g * L, L)
                nv = fold_n[sl]
                sv = fold_s[sl]
                kv = kc + plsc.cumsum(nv)
                Sv = Sc + plsc.cumsum(sv)
                den1 = T + kv - Sv
                denp = den1 - nv + sv
                num = nv * (T - Sv) + kv * sv
                bf = (g * L + lane).astype(jnp.float32)
                mid = (float(K) - 0.5 - bf) * (1.0 / float(K))
                term = mid * num / jnp.maximum(den1 * denp, 1.0)
                return (kc + jnp.sum(nv), Sc + jnp.sum(sv), acc + term)

            _, _, acc = lax.fori_loop(
                0, K // L, sbody,
                (jnp.float32(0.0), jnp.float32(0.0), zeros16))
            accv[...] = accv[...] + acc

    pltpu.sync_copy(accv, out_hbm.at[wid])


@jax.jit
def kernel(input, target):
    t4 = target.astype(jnp.int32).reshape(B, 1, H, W)

    HH = 128
    lse = pl.pallas_call(
        _tc_lse_kernel,
        grid=(B, H // HH),
        in_specs=[pl.BlockSpec((1, C, HH, W), lambda b, j: (b, 0, j, 0))],
        out_specs=pl.BlockSpec((1, 1, HH, W), lambda b, j: (b, 0, j, 0)),
        out_shape=jax.ShapeDtypeStruct((B, 1, H, W), jnp.float32),
    )(input)

    pk = pl.pallas_call(
        _tc_pack_kernel,
        grid=(B, C, W // 128),
        in_specs=[
            pl.BlockSpec((1, 1, H, 128), lambda b, c, jw: (b, c, 0, jw)),
            pl.BlockSpec((1, 1, H, 128), lambda b, c, jw: (b, 0, 0, jw)),
            pl.BlockSpec((1, 1, H, 128), lambda b, c, jw: (b, 0, 0, jw)),
        ],
        out_specs=pl.BlockSpec(
            (H, 128), lambda b, c, jw: ((b * C + c) * (W // 128) + jw, 0)),
        out_shape=jax.ShapeDtypeStruct((NTASK * TROWS, 128), jnp.int16),
    )(input, lse, t4)

    sc_fn = pl.kernel(
        _sc_hist_kernel,
        out_type=jax.ShapeDtypeStruct((NW, L), jnp.float32),
        mesh=plsc.VectorSubcoreMesh(core_axis_name="c", subcore_axis_name="s"),
        compiler_params=pltpu.CompilerParams(needs_layout_passes=False),
        scratch_types=[
            pltpu.VMEM((CROWS, 128), jnp.int16),
            pltpu.VMEM((CROWS, 128), jnp.int16),
            pltpu.VMEM((K * NSLOT,), jnp.float32),
            pltpu.VMEM((K * NSLOT,), jnp.float32),
            pltpu.VMEM((K,), jnp.float32),
            pltpu.VMEM((K,), jnp.float32),
            pltpu.VMEM((L,), jnp.float32),
            pltpu.SemaphoreType.DMA,
            pltpu.SemaphoreType.DMA,
        ],
    )
    partial = sc_fn(pk)
    return jnp.sum(partial) / (B * C)


# speedup vs baseline: 213.8905x; 1.9975x over previous
import jax
import jax.numpy as jnp
from jax import lax
from jax.experimental import pallas as pl
from jax.experimental.pallas import tpu as pltpu
from jax.experimental.pallas import tpu_sc as plsc

B, C, H, W = 4, 19, 512, 512
N = H * W
NTASK = B * C
K = 2048
NSLOT = 16
CH = 16384
CROWS = CH // 128
NCHUNK = N // CH
TROWS = N // 128
NW = 32
L = 16
TASK_ROUNDS = (NTASK + NW - 1) // NW


def _tc_pack_kernel(x_ref, t_ref, pk_ref):
    x = x_ref[...]
    t = t_ref[...]
    m = jnp.max(x, axis=1, keepdims=True)
    ex = jnp.exp(x - m)
    p = ex / jnp.sum(ex, axis=1, keepdims=True)
    cids = lax.broadcasted_iota(jnp.int32, (1, C, 1, 1), 1)
    fg = (t == cids).astype(jnp.float32)
    e = jnp.abs(p - fg)
    qi = ((1.0 - e) * float(K)).astype(jnp.int32)
    qi = jnp.minimum(jnp.maximum(qi, 0), K - 1)
    pk_ref[...] = (qi | (t << 11)).astype(jnp.int16).reshape(C * H, 128)


def _sc_hist_kernel(pk_hbm, out_hbm, pk_buf0, pk_buf1, hist_n, hist_s,
                    fold_n, fold_s, accv, sem0, sem1):
    cid = lax.axis_index("c")
    sid = lax.axis_index("s")
    wid = sid * 2 + cid

    zeros16 = jnp.zeros((L,), jnp.float32)
    ones16 = jnp.ones((L,), jnp.float32)
    lane = lax.broadcasted_iota(jnp.int32, (L,), 0)
    sems = (sem0, sem1)
    accv[...] = zeros16

    for i in range(TASK_ROUNDS):
        tid = wid + NW * i

        @pl.when(tid < NTASK)
        def _():
            def zbody(j, _):
                sl = pl.ds(j * L, L)
                hist_n[sl] = zeros16
                hist_s[sl] = zeros16
                return 0
            lax.fori_loop(0, K * NSLOT // L, zbody, 0)

            bufs = (pk_buf0, pk_buf1)

            def do_chunk(ci):
                pkb = bufs[ci % 2]

                @plsc.parallel_loop(0, CH // 32, unroll=4)
                def _hist(j):
                    r = lax.shift_right_logical(j, 2)
                    cc = j & 3
                    v16 = pkb[r, pl.ds(cc * 32, 32)]
                    v = plsc.bitcast(v16, jnp.int32)
                    for half in range(2):
                        if half == 0:
                            u = v & 0xFFFF
                        else:
                            u = lax.shift_right_logical(v, 16)
                        qi = u & (K - 1)
                        tv = lax.shift_right_logical(u, 11) & 31
                        slot = (qi << 4) | lane
                        plsc.addupdate_scatter(hist_n, [slot], ones16)
                        plsc.addupdate_scatter(hist_s, [slot],
                                               tv.astype(jnp.float32))

            b = ((tid >= C).astype(jnp.int32)
                 + (tid >= 2 * C).astype(jnp.int32)
                 + (tid >= 3 * C).astype(jnp.int32))
            c = tid - b * C

            def chunk_row(ci):
                jw, sub = ci // 4, ci % 4
                return ((b * 4 + jw) * C + c) * 512 + sub * 128

            cp = pltpu.async_copy(
                pk_hbm.at[pl.ds(chunk_row(0), CROWS), :], pk_buf0, sem0)
            for ci in range(NCHUNK):
                if ci + 1 < NCHUNK:
                    nxt = pltpu.async_copy(
                        pk_hbm.at[pl.ds(chunk_row(ci + 1), CROWS), :],
                        (pk_buf0, pk_buf1)[(ci + 1) % 2],
                        sems[(ci + 1) % 2])
                cp.wait()
                do_chunk(ci)
                if ci + 1 < NCHUNK:
                    cp = nxt

            def fbody(g, tacc):
                base = ((g * L + lane) << 4)
                nacc = zeros16
                sacc = zeros16
                for sub in range(NSLOT):
                    idx = base + ((lane + sub) & (NSLOT - 1))
                    nacc = nacc + plsc.load_gather(hist_n, [idx])
                    sacc = sacc + plsc.load_gather(hist_s, [idx])
                sl = pl.ds(g * L, L)
                fold_n[sl] = nacc
                fold_s[sl] = sacc
                return tacc + sacc
            tacc = lax.fori_loop(0, K // L, fbody, zeros16)
            T = jnp.sum(tacc)

            def sbody(g, carry):
                kc, Sc, acc = carry
                sl = pl.ds(g * L, L)
                nv = fold_n[sl]
                sv = fold_s[sl]
                kv = kc + plsc.cumsum(nv)
                Sv = Sc + plsc.cumsum(sv)
                den1 = T + kv - Sv
                denp = den1 - nv + sv
                num = nv * (T - Sv) + kv * sv
                bf = (g * L + lane).astype(jnp.float32)
                mid = (float(K) - 0.5 - bf) * (1.0 / float(K))
                term = mid * num / jnp.maximum(den1 * denp, 1.0)
                return (kc + jnp.sum(nv), Sc + jnp.sum(sv), acc + term)

            _, _, acc = lax.fori_loop(
                0, K // L, sbody,
                (jnp.float32(0.0), jnp.float32(0.0), zeros16))
            accv[...] = accv[...] + acc

    pltpu.sync_copy(accv, out_hbm.at[wid])


@jax.jit
def kernel(input, target):
    t4 = target.astype(jnp.int32).reshape(B, 1, H, W)

    pk = pl.pallas_call(
        _tc_pack_kernel,
        grid=(B, W // 128),
        in_specs=[
            pl.BlockSpec((1, C, H, 128), lambda b, jw: (b, 0, 0, jw)),
            pl.BlockSpec((1, 1, H, 128), lambda b, jw: (b, 0, 0, jw)),
        ],
        out_specs=pl.BlockSpec(
            (C * H, 128), lambda b, jw: (b * (W // 128) + jw, 0)),
        out_shape=jax.ShapeDtypeStruct((NTASK * TROWS, 128), jnp.int16),
    )(input, t4)

    sc_fn = pl.kernel(
        _sc_hist_kernel,
        out_type=jax.ShapeDtypeStruct((NW, L), jnp.float32),
        mesh=plsc.VectorSubcoreMesh(core_axis_name="c", subcore_axis_name="s"),
        compiler_params=pltpu.CompilerParams(needs_layout_passes=False),
        scratch_types=[
            pltpu.VMEM((CROWS, 128), jnp.int16),
            pltpu.VMEM((CROWS, 128), jnp.int16),
            pltpu.VMEM((K * NSLOT,), jnp.float32),
            pltpu.VMEM((K * NSLOT,), jnp.float32),
            pltpu.VMEM((K,), jnp.float32),
            pltpu.VMEM((K,), jnp.float32),
            pltpu.VMEM((L,), jnp.float32),
            pltpu.SemaphoreType.DMA,
            pltpu.SemaphoreType.DMA,
        ],
    )
    partial = sc_fn(pk)
    return jnp.sum(partial) / (B * C)


# speedup vs baseline: 243.7369x; 1.1395x over previous
import jax
import jax.numpy as jnp
from jax import lax
from jax.experimental import pallas as pl
from jax.experimental.pallas import tpu as pltpu
from jax.experimental.pallas import tpu_sc as plsc

B, C, H, W = 4, 19, 512, 512
N = H * W
NTASK = B * C
K = 2048
NSLOT = 16
CH = 16384
CROWS = CH // 128
NCHUNK = N // CH
TROWS = N // 128
NW = 32
L = 16
TASK_ROUNDS = (NTASK + NW - 1) // NW


def _tc_pack_kernel(x_ref, t_ref, pk_ref):
    x = x_ref[...]
    t = t_ref[...]
    m = jnp.max(x, axis=1, keepdims=True)
    ex = jnp.exp(x - m)
    p = ex / jnp.sum(ex, axis=1, keepdims=True)
    cids = lax.broadcasted_iota(jnp.int32, (1, C, 1, 1), 1)
    fg = (t == cids).astype(jnp.float32)
    e = jnp.abs(p - fg)
    qi = ((1.0 - e) * float(K)).astype(jnp.int32)
    qi = jnp.minimum(jnp.maximum(qi, 0), K - 1)
    pk_ref[...] = (qi | (t << 11)).astype(jnp.int16).reshape(C * H, 128)


def _sc_hist_kernel(pk_hbm, out_hbm, pk_buf0, pk_buf1, hist_c,
                    fold_n, fold_s, accv, sem0, sem1):
    cid = lax.axis_index("c")
    sid = lax.axis_index("s")
    wid = sid * 2 + cid

    zeros16 = jnp.zeros((L,), jnp.float32)
    zeros16i = jnp.zeros((L,), jnp.int32)
    lane = lax.broadcasted_iota(jnp.int32, (L,), 0)
    sems = (sem0, sem1)
    accv[...] = zeros16

    for i in range(TASK_ROUNDS):
        tid = wid + NW * i

        @pl.when(tid < NTASK)
        def _():
            def zbody(j, _):
                hist_c[pl.ds(j * L, L)] = zeros16i
                return 0
            lax.fori_loop(0, K * NSLOT // L, zbody, 0)

            bufs = (pk_buf0, pk_buf1)

            def do_chunk(ci):
                pkb = bufs[ci % 2]

                @plsc.parallel_loop(0, CH // 32, unroll=4)
                def _hist(j):
                    r = lax.shift_right_logical(j, 2)
                    cc = j & 3
                    v16 = pkb[r, pl.ds(cc * 32, 32)]
                    v = plsc.bitcast(v16, jnp.int32)
                    for half in range(2):
                        if half == 0:
                            u = v & 0xFFFF
                        else:
                            u = lax.shift_right_logical(v, 16)
                        qi = u & (K - 1)
                        tv = lax.shift_right_logical(u, 11) & 31
                        slot = (qi << 4) | lane
                        plsc.addupdate_scatter(hist_c, [slot],
                                               1 + (tv << 15))

            b = ((tid >= C).astype(jnp.int32)
                 + (tid >= 2 * C).astype(jnp.int32)
                 + (tid >= 3 * C).astype(jnp.int32))
            c = tid - b * C

            def chunk_row(ci):
                jw, sub = ci // 4, ci % 4
                return ((b * 4 + jw) * C + c) * 512 + sub * 128

            cp = pltpu.async_copy(
                pk_hbm.at[pl.ds(chunk_row(0), CROWS), :], pk_buf0, sem0)
            for ci in range(NCHUNK):
                if ci + 1 < NCHUNK:
                    nxt = pltpu.async_copy(
                        pk_hbm.at[pl.ds(chunk_row(ci + 1), CROWS), :],
                        (pk_buf0, pk_buf1)[(ci + 1) % 2],
                        sems[(ci + 1) % 2])
                cp.wait()
                do_chunk(ci)
                if ci + 1 < NCHUNK:
                    cp = nxt

            def fbody(g, tacc):
                base = ((g * L + lane) << 4)
                nacc = zeros16i
                sacc = zeros16i
                for sub in range(NSLOT):
                    idx = base + ((lane + sub) & (NSLOT - 1))
                    comb = plsc.load_gather(hist_c, [idx])
                    nacc = nacc + (comb & 32767)
                    sacc = sacc + lax.shift_right_logical(comb, 15)
                sl = pl.ds(g * L, L)
                fn = nacc.astype(jnp.float32)
                fs = sacc.astype(jnp.float32)
                fold_n[sl] = fn
                fold_s[sl] = fs
                return tacc + fs
            tacc = lax.fori_loop(0, K // L, fbody, zeros16)
            T = jnp.sum(tacc)

            def sbody(g, carry):
                kc, Sc, acc = carry
                sl = pl.ds(g * L, L)
                nv = fold_n[sl]
                sv = fold_s[sl]
                kv = kc + plsc.cumsum(nv)
                Sv = Sc + plsc.cumsum(sv)
                den1 = T + kv - Sv
                denp = den1 - nv + sv
                num = nv * (T - Sv) + kv * sv
                bf = (g * L + lane).astype(jnp.float32)
                mid = (float(K) - 0.5 - bf) * (1.0 / float(K))
                term = mid * num / jnp.maximum(den1 * denp, 1.0)
                return (kc + jnp.sum(nv), Sc + jnp.sum(sv), acc + term)

            _, _, acc = lax.fori_loop(
                0, K // L, sbody,
                (jnp.float32(0.0), jnp.float32(0.0), zeros16))
            accv[...] = accv[...] + acc

    pltpu.sync_copy(accv, out_hbm.at[wid])


@jax.jit
def kernel(input, target):
    t4 = target.astype(jnp.int32).reshape(B, 1, H, W)

    pk = pl.pallas_call(
        _tc_pack_kernel,
        grid=(B, W // 128),
        in_specs=[
            pl.BlockSpec((1, C, H, 128), lambda b, jw: (b, 0, 0, jw)),
            pl.BlockSpec((1, 1, H, 128), lambda b, jw: (b, 0, 0, jw)),
        ],
        out_specs=pl.BlockSpec(
            (C * H, 128), lambda b, jw: (b * (W // 128) + jw, 0)),
        out_shape=jax.ShapeDtypeStruct((NTASK * TROWS, 128), jnp.int16),
    )(input, t4)

    sc_fn = pl.kernel(
        _sc_hist_kernel,
        out_type=jax.ShapeDtypeStruct((NW, L), jnp.float32),
        mesh=plsc.VectorSubcoreMesh(core_axis_name="c", subcore_axis_name="s"),
        compiler_params=pltpu.CompilerParams(needs_layout_passes=False),
        scratch_types=[
            pltpu.VMEM((CROWS, 128), jnp.int16),
            pltpu.VMEM((CROWS, 128), jnp.int16),
            pltpu.VMEM((K * NSLOT,), jnp.int32),
            pltpu.VMEM((K,), jnp.float32),
            pltpu.VMEM((K,), jnp.float32),
            pltpu.VMEM((L,), jnp.float32),
            pltpu.SemaphoreType.DMA,
            pltpu.SemaphoreType.DMA,
        ],
    )
    partial = sc_fn(pk)
    return jnp.sum(partial) / (B * C)
